# Initial kernel scaffold; baseline (speedup 1.0000x reference)
import jax
import jax.numpy as jnp
from jax.experimental import pallas as pl


def kernel(x, edge_index, W1, att_src1, att_dst1, b1, W2, att_src2, att_dst2, b2):
    raise NotImplementedError("write your pallas kernel here")


# speedup vs baseline: 1.0285x; 1.0285x over previous
import functools

import jax
import jax.numpy as jnp
from jax.experimental import pallas as pl
from jax.experimental.pallas import tpu as pltpu

N_NODES = 10000
HEADS = 8
HID_C = 64
OUT_C = 64
NEG_SLOPE = 0.2


def _mm_kernel(x_ref, w_ref, o_ref):
    o_ref[...] = jnp.dot(x_ref[...], w_ref[...],
                         preferred_element_type=jnp.float32)


def _matmul(x, w):
    m, k = x.shape
    k2, n = w.shape
    bm = 1024
    grid = (pl.cdiv(m, bm),)
    return pl.pallas_call(
        _mm_kernel,
        grid=grid,
        in_specs=[pl.BlockSpec((bm, k), lambda i: (i, 0)),
                  pl.BlockSpec((k, n), lambda i: (0, 0))],
        out_specs=pl.BlockSpec((bm, n), lambda i: (i, 0)),
        out_shape=jax.ShapeDtypeStruct((m, n), jnp.float32),
    )(x, w)


def _gat_layer(x, src, dst, W, att_src, att_dst, bias, heads, out_c, concat, n):
    h = _matmul(x, W).reshape(n, heads, out_c)
    alpha_s = (h * att_src).sum(-1)
    alpha_d = (h * att_dst).sum(-1)
    alpha = alpha_s[src] + alpha_d[dst]
    alpha = jnp.where(alpha > 0, alpha, NEG_SLOPE * alpha)
    amax = jax.ops.segment_max(alpha, dst, num_segments=n)
    alpha = jnp.exp(alpha - amax[dst])
    denom = jax.ops.segment_sum(alpha, dst, num_segments=n)
    alpha = alpha / (denom[dst] + 1e-16)
    msg = h[src] * alpha[:, :, None]
    out = jax.ops.segment_sum(msg, dst, num_segments=n)
    if concat:
        out = out.reshape(n, heads * out_c)
    else:
        out = out.mean(axis=1)
    return out + bias


def kernel(x, edge_index, W1, att_src1, att_dst1, b1, W2, att_src2,
           att_dst2, b2):
    n = x.shape[0]
    loop = jnp.arange(n, dtype=edge_index.dtype)
    src = jnp.concatenate([edge_index[0], loop])
    dst = jnp.concatenate([edge_index[1], loop])
    h = _gat_layer(x, src, dst, W1, att_src1, att_dst1, b1, HEADS, HID_C,
                   True, n)
    h = jax.nn.elu(h)
    out = _gat_layer(h, src, dst, W2, att_src2, att_dst2, b2, 1, OUT_C,
                     False, n)
    return out


# speedup vs baseline: 15.4856x; 15.0567x over previous
import functools

import jax
import jax.numpy as jnp
from jax import lax
from jax.experimental import pallas as pl
from jax.experimental.pallas import tpu as pltpu
from jax.experimental.pallas import tpu_sc as plsc

N_NODES = 10000
NP = 10240
IN_C = 128
HID_C = 64
OUT_C = 64
HEADS = 8
NEG_SLOPE = 0.2
E_RAW = 320000
E_TOT = E_RAW + N_NODES
EP = 331776
EROWS = EP // 128
ROWS_L1 = EROWS // 16
ROWS_L2 = EROWS // 32
NB = NP // 1024

_mesh = plsc.VectorSubcoreMesh(
    core_axis_name="c", subcore_axis_name="s", num_cores=2, num_subcores=16)
_sc_params = pltpu.CompilerParams(use_tc_tiling_on_sc=False,
                                  needs_layout_passes=False)


def _tcA_body(x_ref, w_ref, as_w_ref, ad_w_ref,
              h_ref, s_ref, d_ref, ms_ref, md_ref, sh_ref):
    i = pl.program_id(0)

    @pl.when(i == 0)
    def _init():
        ms_ref[...] = jnp.full((HEADS, 128), -jnp.inf, jnp.float32)
        md_ref[...] = jnp.full((HEADS, 128), -jnp.inf, jnp.float32)

    hblk = jnp.dot(x_ref[...], w_ref[...], preferred_element_type=jnp.float32)
    for h in range(HEADS):
        hh = hblk[:, h * HID_C:(h + 1) * HID_C]
        h_ref[h] = hh
        s = jnp.sum(hh * as_w_ref[h][None, :], axis=1)
        d = jnp.sum(hh * ad_w_ref[h][None, :], axis=1)
        s_ref[h] = s
        d_ref[h] = d
        ms_ref[h] = jnp.maximum(ms_ref[h], jnp.full((128,), jnp.max(s)))
        md_ref[h] = jnp.maximum(md_ref[h], jnp.full((128,), jnp.max(d)))

    @pl.when(i == NB - 1)
    def _fin():
        t = ms_ref[...] + md_ref[...]
        t = jnp.where(t > 0, t, NEG_SLOPE * t)
        sh_ref[...] = jnp.maximum(t, 0.0)


def _tcA(xp, W1, as_w, ad_w):
    f32 = jnp.float32
    return pl.pallas_call(
        _tcA_body,
        grid=(NB,),
        in_specs=[
            pl.BlockSpec((1024, IN_C), lambda i: (i, 0)),
            pl.BlockSpec((IN_C, HEADS * HID_C), lambda i: (0, 0)),
            pl.BlockSpec((HEADS, HID_C), lambda i: (0, 0)),
            pl.BlockSpec((HEADS, HID_C), lambda i: (0, 0)),
        ],
        out_specs=[
            pl.BlockSpec((HEADS, 1024, HID_C), lambda i: (0, i, 0)),
            pl.BlockSpec((HEADS, 1024), lambda i: (0, i)),
            pl.BlockSpec((HEADS, 1024), lambda i: (0, i)),
            pl.BlockSpec((HEADS, 128), lambda i: (0, 0)),
            pl.BlockSpec((HEADS, 128), lambda i: (0, 0)),
            pl.BlockSpec((HEADS, 128), lambda i: (0, 0)),
        ],
        out_shape=[
            jax.ShapeDtypeStruct((HEADS, NP, HID_C), f32),
            jax.ShapeDtypeStruct((HEADS, NP), f32),
            jax.ShapeDtypeStruct((HEADS, NP), f32),
            jax.ShapeDtypeStruct((HEADS, 128), f32),
            jax.ShapeDtypeStruct((HEADS, 128), f32),
            jax.ShapeDtypeStruct((HEADS, 128), f32),
        ],
    )(xp, W1, as_w, ad_w)


def _sc1_body(srcr, dstr, asp, adp, shp, h1f,
              out_o, out_d,
              vm_src, vm_dst, vm_as, vm_ad, vm_sh, vm_h, vm_m, vm_e, vm_ix,
              zb, zd, spm_o, spm_d, gsem):
    core = lax.axis_index("c")
    sub = lax.axis_index("s")
    w0 = sub * 640

    def _zrow(r, c):
        for q in range(4):
            zb[r, pl.ds(q * 16, 16)] = jnp.zeros((16,), jnp.float32)
        return c

    lax.fori_loop(0, 128, _zrow, 0)

    def _zdrow(r, c):
        zd[pl.ds(r * 16, 16)] = jnp.zeros((16,), jnp.float32)
        return c

    lax.fori_loop(0, 40, _zdrow, 0)

    pltpu.sync_copy(srcr.at[sub], vm_src)
    pltpu.sync_copy(dstr.at[sub], vm_dst)

    for hh in range(4):
        head = core * 4 + hh
        hoff = pl.multiple_of(head * NP, 128)
        pltpu.sync_copy(asp.at[pl.ds(hoff, NP)], vm_as)
        pltpu.sync_copy(adp.at[pl.ds(hoff, NP)], vm_ad)
        pltpu.sync_copy(shp.at[pl.ds(pl.multiple_of(head * 128, 128), 128)],
                        vm_sh)
        for b in range(5):
            pltpu.sync_copy(zb, spm_o.at[pl.ds(w0 + b * 128, 128)])
        pltpu.sync_copy(zd, spm_d.at[pl.ds(w0, 640)])
        plsc.subcore_barrier()

        def _blk(j, c):
            for k in range(8):
                s16 = vm_src[j, pl.ds(k * 16, 16)]
                d16 = vm_dst[j, pl.ds(k * 16, 16)]
                vm_ix[pl.ds(k * 16, 16)] = s16 + hoff
                sv = plsc.load_gather(vm_as, [s16])
                dv = plsc.load_gather(vm_ad, [d16])
                a = sv + dv
                a = jnp.where(a > 0, a, NEG_SLOPE * a)
                vm_e[pl.ds(k * 16, 16)] = jnp.exp(a - vm_sh[pl.ds(0, 16)])
            pltpu.async_copy(h1f.at[vm_ix], vm_h, gsem).wait()

            def _scale(r, cc):
                es = plsc.load_gather(vm_e, [jnp.full((16,), r, jnp.int32)])
                for q in range(4):
                    vm_m[r, pl.ds(q * 16, 16)] = (
                        vm_h[r, pl.ds(q * 16, 16)] * es)
                return cc

            lax.fori_loop(0, 128, _scale, 0)
            pltpu.sync_copy(vm_m, spm_o.at[vm_dst.at[j]], add=True)
            pltpu.sync_copy(vm_e, spm_d.at[vm_dst.at[j]], add=True)
            return c

        lax.fori_loop(0, ROWS_L1, _blk, 0)
        plsc.subcore_barrier()
        pltpu.sync_copy(spm_o.at[pl.ds(w0, 640)],
                        out_o.at[head, pl.ds(w0, 640)])
        pltpu.sync_copy(spm_d.at[pl.ds(w0, 640)],
                        out_d.at[pl.ds(pl.multiple_of(hoff + w0, 128), 640)])
        plsc.subcore_barrier()


def _sc_edges1(srcr, dstr, asp, adp, shp, h1p):
    f32 = jnp.float32
    fn = pl.kernel(
        _sc1_body,
        out_type=[
            jax.ShapeDtypeStruct((HEADS, NP, HID_C), f32),
            jax.ShapeDtypeStruct((HEADS * NP,), f32),
        ],
        mesh=_mesh,
        compiler_params=_sc_params,
        scratch_types=[
            pltpu.VMEM((ROWS_L1, 128), jnp.int32),
            pltpu.VMEM((ROWS_L1, 128), jnp.int32),
            pltpu.VMEM((NP,), f32),
            pltpu.VMEM((NP,), f32),
            pltpu.VMEM((128,), f32),
            pltpu.VMEM((128, HID_C), f32),
            pltpu.VMEM((128, HID_C), f32),
            pltpu.VMEM((128,), f32),
            pltpu.VMEM((128,), jnp.int32),
            pltpu.VMEM((128, HID_C), f32),
            pltpu.VMEM((640,), f32),
            pltpu.VMEM_SHARED((NP, HID_C), f32),
            pltpu.VMEM_SHARED((NP,), f32),
            pltpu.SemaphoreType.DMA,
        ],
    )
    return fn(srcr, dstr, asp, adp, shp, h1p)


def _tcC_body(p_ref, d_ref, b1_ref, w2_ref, as2_w_ref, ad2_w_ref,
              h2_ref, s2_ref, d2_ref, ms_ref, md_ref, sh_ref):
    i = pl.program_id(0)

    @pl.when(i == 0)
    def _init():
        ms_ref[...] = jnp.full((128,), -jnp.inf, jnp.float32)
        md_ref[...] = jnp.full((128,), -jnp.inf, jnp.float32)

    acc = jnp.zeros((1024, OUT_C), jnp.float32)
    for h in range(HEADS):
        v = p_ref[h] / (d_ref[h][:, None] + 1e-16) + b1_ref[h][None, :]
        v = jnp.where(v > 0, v, jnp.exp(v) - 1.0)
        acc = acc + jnp.dot(v, w2_ref[h], preferred_element_type=jnp.float32)
    h2_ref[...] = acc
    s2 = jnp.sum(acc * as2_w_ref[0][None, :], axis=1)
    d2 = jnp.sum(acc * ad2_w_ref[0][None, :], axis=1)
    s2_ref[...] = s2
    d2_ref[...] = d2
    ms_ref[...] = jnp.maximum(ms_ref[...], jnp.full((128,), jnp.max(s2)))
    md_ref[...] = jnp.maximum(md_ref[...], jnp.full((128,), jnp.max(d2)))

    @pl.when(i == NB - 1)
    def _fin():
        t = ms_ref[...] + md_ref[...]
        t = jnp.where(t > 0, t, NEG_SLOPE * t)
        sh_ref[...] = jnp.maximum(t, 0.0)


def _tcC(out1, den1, b1r, w2r, as2_w, ad2_w):
    f32 = jnp.float32
    return pl.pallas_call(
        _tcC_body,
        grid=(NB,),
        in_specs=[
            pl.BlockSpec((HEADS, 1024, HID_C), lambda i: (0, i, 0)),
            pl.BlockSpec((HEADS, 1024), lambda i: (0, i)),
            pl.BlockSpec((HEADS, HID_C), lambda i: (0, 0)),
            pl.BlockSpec((HEADS, HID_C, OUT_C), lambda i: (0, 0, 0)),
            pl.BlockSpec((1, OUT_C), lambda i: (0, 0)),
            pl.BlockSpec((1, OUT_C), lambda i: (0, 0)),
        ],
        out_specs=[
            pl.BlockSpec((1024, OUT_C), lambda i: (i, 0)),
            pl.BlockSpec((1024,), lambda i: (i,)),
            pl.BlockSpec((1024,), lambda i: (i,)),
            pl.BlockSpec((128,), lambda i: (0,)),
            pl.BlockSpec((128,), lambda i: (0,)),
            pl.BlockSpec((128,), lambda i: (0,)),
        ],
        out_shape=[
            jax.ShapeDtypeStruct((NP, OUT_C), f32),
            jax.ShapeDtypeStruct((NP,), f32),
            jax.ShapeDtypeStruct((NP,), f32),
            jax.ShapeDtypeStruct((128,), f32),
            jax.ShapeDtypeStruct((128,), f32),
            jax.ShapeDtypeStruct((128,), f32),
        ],
    )(out1, den1, b1r, w2r, as2_w, ad2_w)


def _sc2_body(srcr, dstr, asp, adp, shp, h2p,
              out_o, out_d,
              vm_src, vm_dst, vm_as, vm_ad, vm_sh, vm_h, vm_m, vm_e,
              zb, zd, spm_o, spm_d, gsem):
    core = lax.axis_index("c")
    sub = lax.axis_index("s")
    wid = core * 16 + sub
    w0 = sub * 640

    def _zrow(r, c):
        for q in range(4):
            zb[r, pl.ds(q * 16, 16)] = jnp.zeros((16,), jnp.float32)
        return c

    lax.fori_loop(0, 128, _zrow, 0)

    def _zdrow(r, c):
        zd[pl.ds(r * 16, 16)] = jnp.zeros((16,), jnp.float32)
        return c

    lax.fori_loop(0, 40, _zdrow, 0)

    pltpu.sync_copy(srcr.at[wid], vm_src)
    pltpu.sync_copy(dstr.at[wid], vm_dst)
    pltpu.sync_copy(asp, vm_as)
    pltpu.sync_copy(adp, vm_ad)
    pltpu.sync_copy(shp, vm_sh)
    for b in range(5):
        pltpu.sync_copy(zb, spm_o.at[pl.ds(w0 + b * 128, 128)])
    pltpu.sync_copy(zd, spm_d.at[pl.ds(w0, 640)])
    plsc.subcore_barrier()

    def _blk(j, c):
        for k in range(8):
            s16 = vm_src[j, pl.ds(k * 16, 16)]
            d16 = vm_dst[j, pl.ds(k * 16, 16)]
            sv = plsc.load_gather(vm_as, [s16])
            dv = plsc.load_gather(vm_ad, [d16])
            a = sv + dv
            a = jnp.where(a > 0, a, NEG_SLOPE * a)
            vm_e[pl.ds(k * 16, 16)] = jnp.exp(a - vm_sh[pl.ds(0, 16)])
        pltpu.async_copy(h2p.at[vm_src.at[j]], vm_h, gsem).wait()

        def _scale(r, cc):
            es = plsc.load_gather(vm_e, [jnp.full((16,), r, jnp.int32)])
            for q in range(4):
                vm_m[r, pl.ds(q * 16, 16)] = vm_h[r, pl.ds(q * 16, 16)] * es
            return cc

        lax.fori_loop(0, 128, _scale, 0)
        pltpu.sync_copy(vm_m, spm_o.at[vm_dst.at[j]], add=True)
        pltpu.sync_copy(vm_e, spm_d.at[vm_dst.at[j]], add=True)
        return c

    lax.fori_loop(0, ROWS_L2, _blk, 0)
    plsc.subcore_barrier()
    pltpu.sync_copy(spm_o.at[pl.ds(w0, 640)], out_o.at[core, pl.ds(w0, 640)])
    pltpu.sync_copy(
        spm_d.at[pl.ds(w0, 640)],
        out_d.at[pl.ds(pl.multiple_of(core * NP + w0, 128), 640)])


def _sc_edges2(srcr, dstr, asp, adp, shp, h2p):
    f32 = jnp.float32
    fn = pl.kernel(
        _sc2_body,
        out_type=[
            jax.ShapeDtypeStruct((2, NP, OUT_C), f32),
            jax.ShapeDtypeStruct((2 * NP,), f32),
        ],
        mesh=_mesh,
        compiler_params=_sc_params,
        scratch_types=[
            pltpu.VMEM((ROWS_L2, 128), jnp.int32),
            pltpu.VMEM((ROWS_L2, 128), jnp.int32),
            pltpu.VMEM((NP,), f32),
            pltpu.VMEM((NP,), f32),
            pltpu.VMEM((128,), f32),
            pltpu.VMEM((128, OUT_C), f32),
            pltpu.VMEM((128, OUT_C), f32),
            pltpu.VMEM((128,), f32),
            pltpu.VMEM((128, OUT_C), f32),
            pltpu.VMEM((640,), f32),
            pltpu.VMEM_SHARED((NP, OUT_C), f32),
            pltpu.VMEM_SHARED((NP,), f32),
            pltpu.SemaphoreType.DMA,
        ],
    )
    return fn(srcr, dstr, asp, adp, shp, h2p)


def _tcE_body(p_ref, d_ref, b2_ref, o_ref):
    den = d_ref[0] + d_ref[1]
    o_ref[...] = ((p_ref[0] + p_ref[1]) / (den[:, None] + 1e-16)
                  + b2_ref[0][None, :])


def _tcE(out2, den2, b2r):
    return pl.pallas_call(
        _tcE_body,
        grid=(NB,),
        in_specs=[
            pl.BlockSpec((2, 1024, OUT_C), lambda i: (0, i, 0)),
            pl.BlockSpec((2, 1024), lambda i: (0, i)),
            pl.BlockSpec((1, OUT_C), lambda i: (0, 0)),
        ],
        out_specs=pl.BlockSpec((1024, OUT_C), lambda i: (i, 0)),
        out_shape=jax.ShapeDtypeStruct((NP, OUT_C), jnp.float32),
    )(out2, den2, b2r)


def kernel(x, edge_index, W1, att_src1, att_dst1, b1, W2, att_src2,
           att_dst2, b2):
    n = x.shape[0]
    i32 = jnp.int32
    loop = jnp.arange(n, dtype=i32)
    pad = jnp.full((EP - E_TOT,), NP - 1, i32)
    src = jnp.concatenate([edge_index[0].astype(i32), loop, pad])
    dst = jnp.concatenate([edge_index[1].astype(i32), loop, pad])
    srcr16 = src.reshape(16, ROWS_L1, 128)
    dstr16 = dst.reshape(16, ROWS_L1, 128)
    srcr32 = src.reshape(32, ROWS_L2, 128)
    dstr32 = dst.reshape(32, ROWS_L2, 128)
    xp = jnp.pad(x, ((0, NP - n), (0, 0)))

    h1p, asp, adp, _, _, sh1 = _tcA(xp, W1, att_src1, att_dst1)
    out1, den1 = _sc_edges1(srcr16, dstr16, asp.reshape(HEADS * NP),
                            adp.reshape(HEADS * NP),
                            sh1.reshape(HEADS * 128),
                            h1p.reshape(HEADS * NP, HID_C))
    h2p, as2, ad2, _, _, sh2 = _tcC(out1, den1.reshape(HEADS, NP),
                                    b1.reshape(HEADS, HID_C),
                                    W2.reshape(HEADS, HID_C, OUT_C),
                                    att_src2, att_dst2)
    out2, den2 = _sc_edges2(srcr32, dstr32, as2, ad2, sh2, h2p)
    out = _tcE(out2, den2.reshape(2, NP), b2.reshape(1, OUT_C))
    return out[:n]


# speedup vs baseline: 18.8134x; 1.2149x over previous
import jax
import jax.numpy as jnp
from jax import lax
from jax.experimental import pallas as pl
from jax.experimental.pallas import tpu as pltpu
from jax.experimental.pallas import tpu_sc as plsc

N_NODES = 10000
NP = 10240
IN_C = 128
HID_C = 64
OUT_C = 64
HEADS = 8
NEG_SLOPE = 0.2
E_RAW = 320000
E_TOT = E_RAW + N_NODES
EP = 335872
EROWS = EP // 128
ROWS_L1 = EROWS // 16
ROWS_L2 = EROWS // 32
NB = NP // 1024
HC2 = OUT_C // 2

_mesh = plsc.VectorSubcoreMesh(
    core_axis_name="c", subcore_axis_name="s", num_cores=2, num_subcores=16)
_sc_params = pltpu.CompilerParams(use_tc_tiling_on_sc=False,
                                  needs_layout_passes=False)


def _tcA_body(x_ref, w_ref, as_w_ref, ad_w_ref,
              h_ref, s_ref, d_ref, ms_ref, md_ref, sh_ref):
    i = pl.program_id(0)

    @pl.when(i == 0)
    def _init():
        ms_ref[...] = jnp.full((HEADS, 128), -jnp.inf, jnp.float32)
        md_ref[...] = jnp.full((HEADS, 128), -jnp.inf, jnp.float32)

    hblk = jnp.dot(x_ref[...], w_ref[...], preferred_element_type=jnp.float32)
    for h in range(HEADS):
        hh = hblk[:, h * HID_C:(h + 1) * HID_C]
        h_ref[h, 0] = hh[:, :HC2]
        h_ref[h, 1] = hh[:, HC2:]
        s = jnp.sum(hh * as_w_ref[h][None, :], axis=1)
        d = jnp.sum(hh * ad_w_ref[h][None, :], axis=1)
        s_ref[h] = s
        d_ref[h] = d
        ms_ref[h] = jnp.maximum(ms_ref[h], jnp.full((128,), jnp.max(s)))
        md_ref[h] = jnp.maximum(md_ref[h], jnp.full((128,), jnp.max(d)))

    @pl.when(i == NB - 1)
    def _fin():
        t = ms_ref[...] + md_ref[...]
        t = jnp.where(t > 0, t, NEG_SLOPE * t)
        sh_ref[...] = jnp.maximum(t, 0.0)


def _tcA(xp, W1, as_w, ad_w):
    f32 = jnp.float32
    return pl.pallas_call(
        _tcA_body,
        grid=(NB,),
        in_specs=[
            pl.BlockSpec((1024, IN_C), lambda i: (i, 0)),
            pl.BlockSpec((IN_C, HEADS * HID_C), lambda i: (0, 0)),
            pl.BlockSpec((HEADS, HID_C), lambda i: (0, 0)),
            pl.BlockSpec((HEADS, HID_C), lambda i: (0, 0)),
        ],
        out_specs=[
            pl.BlockSpec((HEADS, 2, 1024, HC2), lambda i: (0, 0, i, 0)),
            pl.BlockSpec((HEADS, 1024), lambda i: (0, i)),
            pl.BlockSpec((HEADS, 1024), lambda i: (0, i)),
            pl.BlockSpec((HEADS, 128), lambda i: (0, 0)),
            pl.BlockSpec((HEADS, 128), lambda i: (0, 0)),
            pl.BlockSpec((HEADS, 128), lambda i: (0, 0)),
        ],
        out_shape=[
            jax.ShapeDtypeStruct((HEADS, 2, NP, HC2), f32),
            jax.ShapeDtypeStruct((HEADS, NP), f32),
            jax.ShapeDtypeStruct((HEADS, NP), f32),
            jax.ShapeDtypeStruct((HEADS, 128), f32),
            jax.ShapeDtypeStruct((HEADS, 128), f32),
            jax.ShapeDtypeStruct((HEADS, 128), f32),
        ],
    )(xp, W1, as_w, ad_w)


def _edge_pass(rows, cq, hoff, vm_src, vm_dst, vm_as, vm_ad, vm_sh, hsrc,
               spm_o, spm_d, sets, emit_denom=True):

    def _eix(j, S):
        h_, m_, e_, es_, ix_, gs_, ms_, ds_ = S
        for k in range(8):
            s16 = vm_src[j, pl.ds(k * 16, 16)]
            d16 = vm_dst[j, pl.ds(k * 16, 16)]
            ix_[pl.ds(k * 16, 16)] = s16 + hoff
            sv = plsc.load_gather(vm_as, [s16])
            dv = plsc.load_gather(vm_ad, [d16])
            a = sv + dv
            a = jnp.where(a > 0, a, NEG_SLOPE * a)
            e_[pl.ds(k * 16, 16)] = jnp.exp(a - vm_sh[pl.ds(0, 16)])
        pltpu.async_copy(hsrc.at[ix_], h_, gs_)

    def _half(j, t, n_t, S):
        h_, m_, e_, es_, ix_, gs_, ms_, ds_ = S

        @pl.when(t > 0)
        def _w():
            pltpu.make_async_copy(m_, spm_o.at[vm_dst.at[j]], ms_).wait()
            if emit_denom:
                pltpu.make_async_copy(es_, spm_d.at[vm_dst.at[j]], ds_).wait()

        pltpu.make_async_copy(hsrc.at[ix_], h_, gs_).wait()

        def _scale(r, c):
            ev = plsc.load_gather(e_, [jnp.full((16,), r, jnp.int32)])
            for q in range(cq):
                m_[r, pl.ds(q * 16, 16)] = h_[r, pl.ds(q * 16, 16)] * ev
            return c

        lax.fori_loop(0, 128, _scale, 0)
        pltpu.async_copy(m_, spm_o.at[vm_dst.at[j]], ms_, add=True)
        if emit_denom:
            for k in range(8):
                es_[pl.ds(k * 16, 16)] = e_[pl.ds(k * 16, 16)]
            pltpu.async_copy(es_, spm_d.at[vm_dst.at[j]], ds_, add=True)

        @pl.when(t < n_t - 1)
        def _p():
            _eix(j + 2, S)

    S0, S1 = sets
    n_t = rows // 2
    _eix(0, S0)
    _eix(1, S1)

    def _body(t, c):
        _half(2 * t, t, n_t, S0)
        _half(2 * t + 1, t, n_t, S1)
        return c

    lax.fori_loop(0, n_t, _body, 0)
    for S in sets:
        h_, m_, e_, es_, ix_, gs_, ms_, ds_ = S
        pltpu.make_async_copy(m_, spm_o.at[vm_dst.at[0]], ms_).wait()
        if emit_denom:
            pltpu.make_async_copy(es_, spm_d.at[vm_dst.at[0]], ds_).wait()


def _zero_bufs(zb, zd, cq=4):
    def _zrow(r, c):
        for q in range(cq):
            zb[r, pl.ds(q * 16, 16)] = jnp.zeros((16,), jnp.float32)
        return c

    lax.fori_loop(0, 128, _zrow, 0)

    def _zdrow(r, c):
        zd[pl.ds(r * 16, 16)] = jnp.zeros((16,), jnp.float32)
        return c

    lax.fori_loop(0, 40, _zdrow, 0)


def _sc1_body(srcr, dstr, asp, adp, shp, h1f,
              out_o, out_d,
              vm_src, vm_dst, vm_as, vm_ad, vm_sh,
              h0, m0, e0, es0, ix0, h1, m1, e1, es1, ix1,
              zb, zd, spm_o, spm_d, gs0, ms0, ds0, gs1, ms1, ds1):
    core = lax.axis_index("c")
    sub = lax.axis_index("s")
    w0 = sub * 640
    sets = ((h0, m0, e0, es0, ix0, gs0, ms0, ds0),
            (h1, m1, e1, es1, ix1, gs1, ms1, ds1))

    _zero_bufs(zb, zd, cq=HC2 // 16)
    pltpu.sync_copy(srcr.at[sub], vm_src)
    pltpu.sync_copy(dstr.at[sub], vm_dst)

    for hh in range(4):
        head = core * 4 + hh
        aoff = pl.multiple_of(head * NP, 128)
        pltpu.sync_copy(asp.at[pl.ds(aoff, NP)], vm_as)
        pltpu.sync_copy(adp.at[pl.ds(aoff, NP)], vm_ad)
        pltpu.sync_copy(shp.at[pl.ds(pl.multiple_of(head * 128, 128), 128)],
                        vm_sh)
        for half in range(2):
            hoff = pl.multiple_of((head * 2 + half) * NP, 128)
            for b in range(5):
                pltpu.sync_copy(zb, spm_o.at[pl.ds(w0 + b * 128, 128)])
            if half == 0:
                pltpu.sync_copy(zd, spm_d.at[pl.ds(w0, 640)])
            plsc.subcore_barrier()
            _edge_pass(ROWS_L1, HC2 // 16, hoff, vm_src, vm_dst, vm_as,
                       vm_ad, vm_sh, h1f, spm_o, spm_d, sets,
                       emit_denom=(half == 0))
            plsc.subcore_barrier()
            pltpu.sync_copy(spm_o.at[pl.ds(w0, 640)],
                            out_o.at[head, half, pl.ds(w0, 640)])
            if half == 0:
                pltpu.sync_copy(
                    spm_d.at[pl.ds(w0, 640)],
                    out_d.at[pl.ds(pl.multiple_of(aoff + w0, 128), 640)])
            plsc.subcore_barrier()


def _sc_edges1(srcr, dstr, asp, adp, shp, h1f):
    f32 = jnp.float32
    i32 = jnp.int32
    bufset = [
        pltpu.VMEM((128, HC2), f32),
        pltpu.VMEM((128, HC2), f32),
        pltpu.VMEM((128,), f32),
        pltpu.VMEM((128,), f32),
        pltpu.VMEM((128,), i32),
    ]
    fn = pl.kernel(
        _sc1_body,
        out_type=[
            jax.ShapeDtypeStruct((HEADS, 2, NP, HC2), f32),
            jax.ShapeDtypeStruct((HEADS * NP,), f32),
        ],
        mesh=_mesh,
        compiler_params=_sc_params,
        scratch_types=[
            pltpu.VMEM((ROWS_L1, 128), i32),
            pltpu.VMEM((ROWS_L1, 128), i32),
            pltpu.VMEM((NP,), f32),
            pltpu.VMEM((NP,), f32),
            pltpu.VMEM((128,), f32),
        ] + bufset + bufset + [
            pltpu.VMEM((128, HC2), f32),
            pltpu.VMEM((640,), f32),
            pltpu.VMEM_SHARED((NP, HC2), f32),
            pltpu.VMEM_SHARED((NP,), f32),
        ] + [pltpu.SemaphoreType.DMA] * 6,
    )
    return fn(srcr, dstr, asp, adp, shp, h1f)


def _tcC_body(p_ref, d_ref, b1_ref, w2_ref, as2_w_ref, ad2_w_ref,
              h2_ref, s2_ref, d2_ref, ms_ref, md_ref, sh_ref):
    i = pl.program_id(0)

    @pl.when(i == 0)
    def _init():
        ms_ref[...] = jnp.full((128,), -jnp.inf, jnp.float32)
        md_ref[...] = jnp.full((128,), -jnp.inf, jnp.float32)

    acc = jnp.zeros((1024, OUT_C), jnp.float32)
    for h in range(HEADS):
        ph = jnp.concatenate((p_ref[h, 0], p_ref[h, 1]), axis=1)
        v = ph / (d_ref[h][:, None] + 1e-16) + b1_ref[h][None, :]
        v = jnp.where(v > 0, v, jnp.exp(v) - 1.0)
        acc = acc + jnp.dot(v, w2_ref[h], preferred_element_type=jnp.float32)
    h2_ref[0] = acc[:, :32]
    h2_ref[1] = acc[:, 32:]
    s2 = jnp.sum(acc * as2_w_ref[0][None, :], axis=1)
    d2 = jnp.sum(acc * ad2_w_ref[0][None, :], axis=1)
    s2_ref[...] = s2
    d2_ref[...] = d2
    ms_ref[...] = jnp.maximum(ms_ref[...], jnp.full((128,), jnp.max(s2)))
    md_ref[...] = jnp.maximum(md_ref[...], jnp.full((128,), jnp.max(d2)))

    @pl.when(i == NB - 1)
    def _fin():
        t = ms_ref[...] + md_ref[...]
        t = jnp.where(t > 0, t, NEG_SLOPE * t)
        sh_ref[...] = jnp.maximum(t, 0.0)


def _tcC(out1, den1, b1r, w2r, as2_w, ad2_w):
    f32 = jnp.float32
    return pl.pallas_call(
        _tcC_body,
        grid=(NB,),
        in_specs=[
            pl.BlockSpec((HEADS, 2, 1024, HC2), lambda i: (0, 0, i, 0)),
            pl.BlockSpec((HEADS, 1024), lambda i: (0, i)),
            pl.BlockSpec((HEADS, HID_C), lambda i: (0, 0)),
            pl.BlockSpec((HEADS, HID_C, OUT_C), lambda i: (0, 0, 0)),
            pl.BlockSpec((1, OUT_C), lambda i: (0, 0)),
            pl.BlockSpec((1, OUT_C), lambda i: (0, 0)),
        ],
        out_specs=[
            pl.BlockSpec((2, 1024, OUT_C // 2), lambda i: (0, i, 0)),
            pl.BlockSpec((1024,), lambda i: (i,)),
            pl.BlockSpec((1024,), lambda i: (i,)),
            pl.BlockSpec((128,), lambda i: (0,)),
            pl.BlockSpec((128,), lambda i: (0,)),
            pl.BlockSpec((128,), lambda i: (0,)),
        ],
        out_shape=[
            jax.ShapeDtypeStruct((2, NP, OUT_C // 2), f32),
            jax.ShapeDtypeStruct((NP,), f32),
            jax.ShapeDtypeStruct((NP,), f32),
            jax.ShapeDtypeStruct((128,), f32),
            jax.ShapeDtypeStruct((128,), f32),
            jax.ShapeDtypeStruct((128,), f32),
        ],
    )(out1, den1, b1r, w2r, as2_w, ad2_w)


def _sc2_body(srcr, dstr, asp, adp, shp, h2f,
              out_o, out_d,
              vm_src, vm_dst, vm_as, vm_ad, vm_sh,
              h0, m0, e0, es0, ix0, h1, m1, e1, es1, ix1,
              zb, zd, spm_o, spm_d, gs0, ms0, ds0, gs1, ms1, ds1):
    core = lax.axis_index("c")
    sub = lax.axis_index("s")
    w0 = sub * 640
    sets = ((h0, m0, e0, es0, ix0, gs0, ms0, ds0),
            (h1, m1, e1, es1, ix1, gs1, ms1, ds1))

    _zero_bufs(zb, zd, cq=HC2 // 16)
    pltpu.sync_copy(srcr.at[sub], vm_src)
    pltpu.sync_copy(dstr.at[sub], vm_dst)
    pltpu.sync_copy(asp, vm_as)
    pltpu.sync_copy(adp, vm_ad)
    pltpu.sync_copy(shp, vm_sh)
    for b in range(5):
        pltpu.sync_copy(zb, spm_o.at[pl.ds(w0 + b * 128, 128)])
    pltpu.sync_copy(zd, spm_d.at[pl.ds(w0, 640)])
    plsc.subcore_barrier()
    hoff = pl.multiple_of(core * NP, 128)
    _edge_pass(ROWS_L1, HC2 // 16, hoff, vm_src, vm_dst, vm_as, vm_ad,
               vm_sh, h2f, spm_o, spm_d, sets)
    plsc.subcore_barrier()
    pltpu.sync_copy(spm_o.at[pl.ds(w0, 640)], out_o.at[core, pl.ds(w0, 640)])

    @pl.when(core == 0)
    def _dd():
        pltpu.sync_copy(spm_d.at[pl.ds(w0, 640)], out_d.at[pl.ds(w0, 640)])


def _sc_edges2(srcr, dstr, asp, adp, shp, h2f):
    f32 = jnp.float32
    i32 = jnp.int32
    bufset = [
        pltpu.VMEM((128, HC2), f32),
        pltpu.VMEM((128, HC2), f32),
        pltpu.VMEM((128,), f32),
        pltpu.VMEM((128,), f32),
        pltpu.VMEM((128,), i32),
    ]
    fn = pl.kernel(
        _sc2_body,
        out_type=[
            jax.ShapeDtypeStruct((2, NP, HC2), f32),
            jax.ShapeDtypeStruct((NP,), f32),
        ],
        mesh=_mesh,
        compiler_params=_sc_params,
        scratch_types=[
            pltpu.VMEM((ROWS_L1, 128), i32),
            pltpu.VMEM((ROWS_L1, 128), i32),
            pltpu.VMEM((NP,), f32),
            pltpu.VMEM((NP,), f32),
            pltpu.VMEM((128,), f32),
        ] + bufset + bufset + [
            pltpu.VMEM((128, HC2), f32),
            pltpu.VMEM((640,), f32),
            pltpu.VMEM_SHARED((NP, HC2), f32),
            pltpu.VMEM_SHARED((NP,), f32),
        ] + [pltpu.SemaphoreType.DMA] * 6,
    )
    return fn(srcr, dstr, asp, adp, shp, h2f)


def _tcE_body(p_ref, d_ref, b2_ref, o_ref):
    den = d_ref[...]
    full = jnp.concatenate((p_ref[0], p_ref[1]), axis=1)
    o_ref[...] = full / (den[:, None] + 1e-16) + b2_ref[0][None, :]


def _tcE(out2, den2, b2r):
    return pl.pallas_call(
        _tcE_body,
        grid=(NB,),
        in_specs=[
            pl.BlockSpec((2, 1024, HC2), lambda i: (0, i, 0)),
            pl.BlockSpec((1024,), lambda i: (i,)),
            pl.BlockSpec((1, OUT_C), lambda i: (0, 0)),
        ],
        out_specs=pl.BlockSpec((1024, OUT_C), lambda i: (i, 0)),
        out_shape=jax.ShapeDtypeStruct((NP, OUT_C), jnp.float32),
    )(out2, den2, b2r)


def kernel(x, edge_index, W1, att_src1, att_dst1, b1, W2, att_src2,
           att_dst2, b2):
    n = x.shape[0]
    i32 = jnp.int32
    loop = jnp.arange(n, dtype=i32)
    pad = jnp.full((EP - E_TOT,), NP - 1, i32)
    src = jnp.concatenate([edge_index[0].astype(i32), loop, pad])
    dst = jnp.concatenate([edge_index[1].astype(i32), loop, pad])
    srcr16 = src.reshape(16, ROWS_L1, 128)
    dstr16 = dst.reshape(16, ROWS_L1, 128)
    xp = jnp.pad(x, ((0, NP - n), (0, 0)))

    h1p, asp, adp, _, _, sh1 = _tcA(xp, W1, att_src1, att_dst1)
    out1, den1 = _sc_edges1(srcr16, dstr16, asp.reshape(HEADS * NP),
                            adp.reshape(HEADS * NP),
                            sh1.reshape(HEADS * 128),
                            h1p.reshape(HEADS * 2 * NP, HC2))
    h2s, as2, ad2, _, _, sh2 = _tcC(out1, den1.reshape(HEADS, NP),
                                    b1.reshape(HEADS, HID_C),
                                    W2.reshape(HEADS, HID_C, OUT_C),
                                    att_src2, att_dst2)
    out2, den2 = _sc_edges2(srcr16, dstr16, as2, ad2, sh2,
                            h2s.reshape(2 * NP, HC2))
    out = _tcE(out2, den2, b2.reshape(1, OUT_C))
    return out[:n]
